# Initial kernel scaffold; baseline (speedup 1.0000x reference)
import jax
import jax.numpy as jnp
from jax.experimental import pallas as pl


def kernel(x, edge_index, W_l, W_r, att, bias_gat, W_ih, W_hh, b_ih, b_hh, W_fc, b_fc, log_std):
    raise NotImplementedError("write your pallas kernel here")


# speedup vs baseline: 12.6708x; 12.6708x over previous
import functools

import jax
import jax.numpy as jnp
from jax import lax
from jax.experimental import pallas as pl
from jax.experimental.pallas import tpu as pltpu
from jax.experimental.pallas import tpu_sc as plsc

N = 10000
F = 128
NH = 4
HC = 64
HH = NH * HC
E = 160000
GH = 128
G3 = 3 * GH

NC, NS, LANES = 2, 16, 16


def _permute(v, idx):
    return lax.gather(
        v, idx[:, None],
        dimension_numbers=lax.GatherDimensionNumbers(
            offset_dims=(), collapsed_slice_dims=(0,), start_index_map=(0,)),
        slice_sizes=(1,),
        mode=lax.GatherScatterMode.PROMISE_IN_BOUNDS)


def _lane_total(v, lane):
    for sh in (8, 4, 2, 1):
        v = v + _permute(v, jnp.bitwise_xor(lane, sh))
    return v
HALF = HH // NC
CB = 80
ROWS_PER_TILE = 632
NPAD = NS * ROWS_PER_TILE
DROWS = 256
DR_PER_TILE = DROWS // NS


def _proj_body(x_ref, wl_ref, wr_ref, xlh_ref, xrh_ref):
    xb = x_ref[...]
    xl = jnp.dot(xb, wl_ref[...], preferred_element_type=jnp.float32)
    xr = jnp.dot(xb, wr_ref[...], preferred_element_type=jnp.float32)
    xlh_ref[0, :, :] = xl[:, :HALF]
    xlh_ref[1, :, :] = xl[:, HALF:]
    xrh_ref[0, :, :] = xr[:, :HALF]
    xrh_ref[1, :, :] = xr[:, HALF:]


def _proj(x, W_l, W_r):
    BN = 1000
    return pl.pallas_call(
        _proj_body,
        grid=(N // BN,),
        in_specs=[pl.BlockSpec((BN, F), lambda i: (i, 0)),
                  pl.BlockSpec((F, HH), lambda i: (0, 0)),
                  pl.BlockSpec((F, HH), lambda i: (0, 0))],
        out_specs=[pl.BlockSpec((NC, BN, HALF), lambda i: (0, i, 0)),
                   pl.BlockSpec((NC, BN, HALF), lambda i: (0, i, 0))],
        out_shape=[jax.ShapeDtypeStruct((NC, N, HALF), jnp.float32),
                   jax.ShapeDtypeStruct((NC, N, HALF), jnp.float32)],
    )(x, W_l, W_r)


def _sc_body(xlf, xrf, src_hbm, dst_hbm, att_hbm, zer_hbm, u_hbm, d_hbm,
             srcb, dstb, dpad, dpa2, dadj, d8b, xlb, xrb, stage, staged,
             attb, U, Dp, sem):
    c = lax.axis_index("c")
    s = lax.axis_index("s")
    pltpu.sync_copy(att_hbm.at[c], attb)
    pltpu.sync_copy(zer_hbm, U.at[pl.ds(s * ROWS_PER_TILE, ROWS_PER_TILE)])
    pltpu.sync_copy(zer_hbm.at[pl.ds(0, DR_PER_TILE)],
                    Dp.at[pl.ds(s * DR_PER_TILE, DR_PER_TILE)])
    plsc.subcore_barrier()

    ec = E // NS
    e0 = s * ec
    coff = c * N

    def chunk(g, carry):
        off = e0 + g * CB
        pltpu.sync_copy(src_hbm.at[pl.ds(off, CB)], srcb.at[0])
        pltpu.sync_copy(dst_hbm.at[pl.ds(off, CB)], dstb.at[0])
        for k in range(CB // LANES):
            sl = pl.ds(k * LANES, LANES)
            srcb[0, sl] = srcb[0, sl] + coff
            dv = dstb[0, sl]
            dpad[0, sl] = (dv & 7) * 2
            dpa2[0, sl] = lax.shift_right_logical(dv, 3) & 7
            dadj[0, sl] = dv + coff
            d8b[0, sl] = lax.shift_right_logical(dv, 6)
        g1 = pltpu.async_copy(xlf.at[srcb.at[0]], xlb, sem)
        g2 = pltpu.async_copy(xrf.at[dadj.at[0]], xrb, sem)
        g1.wait()
        g2.wait()

        def edge(i, carry2):
            vls = []
            acc0 = None
            acc1 = None
            for k in range(8):
                sl = pl.ds(k * LANES, LANES)
                vl = xlb[i, sl]
                vr = xrb[i, sl]
                vls.append(vl)
                sm = vl + vr
                ep = jnp.maximum(sm, 0.2 * sm)
                p = ep * attb[k]
                if k < 4:
                    acc0 = p if acc0 is None else acc0 + p
                else:
                    acc1 = p if acc1 is None else acc1 + p
            lane = lax.iota(jnp.int32, LANES)
            ea0 = jnp.exp(_lane_total(acc0, lane))
            ea1 = jnp.exp(_lane_total(acc1, lane))
            for k in range(8):
                w = ea0 if k < 4 else ea1
                stage[i, pl.ds(k * LANES, LANES)] = vls[k] * w
            m16 = dpad[0, pl.ds(i, LANES)]
            m16b = dpa2[0, pl.ds(i, LANES)]
            la = jnp.zeros((LANES,), jnp.int32) + m16[0]
            mv = jnp.zeros((LANES,), jnp.int32) + m16b[0]
            mk0 = (1 - jnp.minimum(jnp.abs(lane - la), 1)).astype(
                jnp.float32)
            mk1 = (1 - jnp.minimum(jnp.abs(lane - la - 1), 1)).astype(
                jnp.float32)
            eav = ea0 * mk0 + ea1 * mk1
            for k in range(8):
                maskf = (1 - jnp.minimum(jnp.abs(mv - k), 1)).astype(
                    jnp.float32)
                staged[i, pl.ds(k * LANES, LANES)] = eav * maskf
            return carry2

        lax.fori_loop(0, CB, edge, 0)
        pltpu.sync_copy(stage, U.at[dstb.at[0]], add=True)
        pltpu.sync_copy(staged, Dp.at[d8b.at[0]], add=True)
        return carry

    lax.fori_loop(0, ec // CB, chunk, 0)
    plsc.subcore_barrier()
    r0 = s * ROWS_PER_TILE
    pltpu.sync_copy(U.at[pl.ds(r0, ROWS_PER_TILE)],
                    u_hbm.at[c, pl.ds(r0, ROWS_PER_TILE)])
    d0 = s * DR_PER_TILE
    pltpu.sync_copy(Dp.at[pl.ds(d0, DR_PER_TILE)],
                    d_hbm.at[c, pl.ds(d0, DR_PER_TILE)])


def _edge_sc(xlh, xrh, src, dst, att_sc, zer):
    mesh = plsc.VectorSubcoreMesh(core_axis_name="c", subcore_axis_name="s",
                                  num_cores=NC, num_subcores=NS)
    kfn = pl.kernel(
        _sc_body,
        out_type=[jax.ShapeDtypeStruct((NC, NPAD, HALF), jnp.float32),
                  jax.ShapeDtypeStruct((NC, DROWS, HALF), jnp.float32)],
        mesh=mesh,
        scratch_types=[
            pltpu.VMEM((1, CB), jnp.int32),
            pltpu.VMEM((1, CB), jnp.int32),
            pltpu.VMEM((1, CB + LANES), jnp.int32),
            pltpu.VMEM((1, CB + LANES), jnp.int32),
            pltpu.VMEM((1, CB), jnp.int32),
            pltpu.VMEM((1, CB), jnp.int32),
            pltpu.VMEM((CB, HALF), jnp.float32),
            pltpu.VMEM((CB, HALF), jnp.float32),
            pltpu.VMEM((CB, HALF), jnp.float32),
            pltpu.VMEM((CB, HALF), jnp.float32),
            pltpu.VMEM((8, LANES), jnp.float32),
            pltpu.VMEM_SHARED((NPAD, HALF), jnp.float32),
            pltpu.VMEM_SHARED((DROWS, HALF), jnp.float32),
            pltpu.SemaphoreType.DMA,
        ],
    )
    return kfn(xlh.reshape(NC * N, HALF), xrh.reshape(NC * N, HALF),
               src, dst, att_sc, zer)


def _combine_body(usc_ref, d4_ref, xlh_ref, xrh_ref, attf_ref, bias_ref,
                  wih_ref, bih_ref, gi_ref):
    xl = jnp.concatenate([xlh_ref[0], xlh_ref[1]], axis=1)
    xr = jnp.concatenate([xrh_ref[0], xrh_ref[1]], axis=1)
    U = jnp.concatenate([usc_ref[0], usc_ref[1]], axis=1)
    D4 = d4_ref[...]
    sm = xl + xr
    ep = jnp.maximum(sm, 0.2 * sm)
    m = ep * attf_ref[...]
    r_i = lax.broadcasted_iota(jnp.int32, (HH, NH), 0)
    c_i = lax.broadcasted_iota(jnp.int32, (HH, NH), 1)
    hsel = (lax.div(r_i, HC) == c_i).astype(jnp.float32)
    r_t = lax.broadcasted_iota(jnp.int32, (NH, HH), 0)
    c_t = lax.broadcasted_iota(jnp.int32, (NH, HH), 1)
    hselT = (r_t == lax.div(c_t, HC)).astype(jnp.float32)
    a_self = jnp.dot(m, hsel, preferred_element_type=jnp.float32)
    ea_self = jnp.exp(a_self)
    den4 = D4 + ea_self
    ea_full = jnp.dot(ea_self, hselT, preferred_element_type=jnp.float32)
    den_full = jnp.dot(den4, hselT, preferred_element_type=jnp.float32)
    gat = (U + ea_full * xl) / den_full + bias_ref[...]
    gi_ref[...] = (jnp.dot(gat, wih_ref[...],
                           preferred_element_type=jnp.float32) + bih_ref[...])


def _combine(usc, d4, xlh, xrh, attf, bias, wiht, bih):
    BN = 1000
    return pl.pallas_call(
        _combine_body,
        grid=(N // BN,),
        in_specs=[pl.BlockSpec((NC, BN, HALF), lambda i: (0, i, 0)),
                  pl.BlockSpec((BN, NH), lambda i: (i, 0)),
                  pl.BlockSpec((NC, BN, HALF), lambda i: (0, i, 0)),
                  pl.BlockSpec((NC, BN, HALF), lambda i: (0, i, 0)),
                  pl.BlockSpec((1, HH), lambda i: (0, 0)),
                  pl.BlockSpec((1, HH), lambda i: (0, 0)),
                  pl.BlockSpec((HH, G3), lambda i: (0, 0)),
                  pl.BlockSpec((1, G3), lambda i: (0, 0))],
        out_specs=pl.BlockSpec((BN, G3), lambda i: (i, 0)),
        out_shape=jax.ShapeDtypeStruct((N, G3), jnp.float32),
    )(usc, d4, xlh, xrh, attf, bias, wiht, bih)


_BT = 1000
_TB = N // _BT


def _gru_body(gi_ref, whht_ref, bhh_ref, wfc_ref, bfc_ref, ls_ref,
              mean_ref, std_ref, h_ref):
    t = pl.program_id(0)

    @pl.when(t == 0)
    def _init():
        h_ref[...] = jnp.zeros_like(h_ref)

    whht = whht_ref[...]
    bhh = bhh_ref[...]

    def step(i, h):
        gh = jnp.dot(h, whht, preferred_element_type=jnp.float32) + bhh
        gi_t = gi_ref[pl.ds(i, 1), :]
        r = jax.nn.sigmoid(gi_t[:, :GH] + gh[:, :GH])
        z = jax.nn.sigmoid(gi_t[:, GH:2 * GH] + gh[:, GH:2 * GH])
        nn_ = jnp.tanh(gi_t[:, 2 * GH:] + r * gh[:, 2 * GH:])
        return (1.0 - z) * nn_ + z * h

    h = lax.fori_loop(0, _BT, step, h_ref[0:1, :])
    h_ref[0:1, :] = h

    @pl.when(t == _TB - 1)
    def _fin():
        mean_ref[...] = (jnp.dot(h, wfc_ref[...],
                                 preferred_element_type=jnp.float32)
                         + bfc_ref[...])
        std_ref[...] = jnp.exp(jnp.clip(ls_ref[...], -10.0, 2.0))


def _gru_head(gi, whht, bhh, wfc, bfc, ls):
    return pl.pallas_call(
        _gru_body,
        grid=(_TB,),
        in_specs=[pl.BlockSpec((_BT, G3), lambda i: (i, 0)),
                  pl.BlockSpec((GH, G3), lambda i: (0, 0)),
                  pl.BlockSpec((1, G3), lambda i: (0, 0)),
                  pl.BlockSpec((GH, N), lambda i: (0, 0)),
                  pl.BlockSpec((1, N), lambda i: (0, 0)),
                  pl.BlockSpec((1, N), lambda i: (0, 0))],
        out_specs=[pl.BlockSpec((1, N), lambda i: (0, 0)),
                   pl.BlockSpec((1, N), lambda i: (0, 0))],
        out_shape=[jax.ShapeDtypeStruct((1, N), jnp.float32),
                   jax.ShapeDtypeStruct((1, N), jnp.float32)],
        scratch_shapes=[pltpu.VMEM((8, GH), jnp.float32)],
    )(gi, whht, bhh, wfc, bfc, ls)


def kernel(x, edge_index, W_l, W_r, att, bias_gat, W_ih, W_hh, b_ih, b_hh,
           W_fc, b_fc, log_std):
    src = edge_index[0].astype(jnp.int32)
    dst = edge_index[1].astype(jnp.int32)
    xlh, xrh = _proj(x, W_l, W_r)
    att_sc = att.reshape(NC, 8, LANES)
    zer = jnp.zeros((ROWS_PER_TILE, HALF), jnp.float32)
    usc, dsc = _edge_sc(xlh, xrh, src, dst, att_sc, zer)
    dflat = dsc.reshape(NC, DROWS * 64, 2)[:, :N]
    d4 = jnp.concatenate([dflat[0], dflat[1]], axis=1)
    gi = _combine(usc, d4, xlh, xrh, att.reshape(1, HH),
                  bias_gat.reshape(1, HH), W_ih.T, b_ih.reshape(1, G3))
    mean, std = _gru_head(gi, W_hh.T, b_hh.reshape(1, G3), W_fc,
                          b_fc.reshape(1, N), log_std.reshape(1, N))
    return mean.reshape(N), std.reshape(N)
